# Initial kernel scaffold; baseline (speedup 1.0000x reference)
import jax
import jax.numpy as jnp
from jax.experimental import pallas as pl


def kernel(x, edge_index, W_pool0, b_pool0, W0, b0, W_pool1, b_pool1, W1, b1, W_ro, b_ro):
    raise NotImplementedError("write your pallas kernel here")


# speedup vs baseline: 1.7045x; 1.7045x over previous
import functools

import jax
import jax.numpy as jnp
from jax import lax
from jax.experimental import pallas as pl
from jax.experimental.pallas import tpu as pltpu
from jax.experimental.pallas import tpu_sc as plsc

N_NODES = 10000
N_EDGES = 160000
D_FEAT = 256
AGG_DIM = 512
RO_DIM = 128

NW = 32
NPT = 320
NPAD = NW * NPT
K = 64
EPAD = N_EDGES + 96
ROW_BLK = 400
SL = AGG_DIM // 16


def _dual_matmul_body(x_ref, wp_ref, bp_ref, wt_ref, m_ref, xw_ref):
    x = x_ref[...]
    m_ref[...] = jax.nn.relu(
        jnp.dot(x, wp_ref[...], preferred_element_type=jnp.float32) + bp_ref[...]
    )
    xw_ref[...] = jnp.dot(x, wt_ref[...], preferred_element_type=jnp.float32)


def _tc_pool_and_top(x, wp, bp, wt):
    n, d = x.shape
    grid = (n // ROW_BLK,)
    return pl.pallas_call(
        _dual_matmul_body,
        grid=grid,
        in_specs=[
            pl.BlockSpec((ROW_BLK, d), lambda i: (i, 0)),
            pl.BlockSpec((d, AGG_DIM), lambda i: (0, 0)),
            pl.BlockSpec((AGG_DIM,), lambda i: (0,)),
            pl.BlockSpec((d, AGG_DIM), lambda i: (0, 0)),
        ],
        out_specs=[
            pl.BlockSpec((ROW_BLK, AGG_DIM), lambda i: (i, 0)),
            pl.BlockSpec((ROW_BLK, AGG_DIM), lambda i: (i, 0)),
        ],
        out_shape=[
            jax.ShapeDtypeStruct((n, AGG_DIM), jnp.float32),
            jax.ShapeDtypeStruct((n, AGG_DIM), jnp.float32),
        ],
    )(x, wp, bp, wt)


def _mid_body(agg_ref, xw_ref, wb_ref, b_ref, wp_ref, bp_ref, wt_ref, m_ref, hw_ref):
    h = jax.nn.relu(
        xw_ref[...]
        + jnp.dot(agg_ref[...], wb_ref[...], preferred_element_type=jnp.float32)
        + b_ref[...]
    )
    m_ref[...] = jax.nn.relu(
        jnp.dot(h, wp_ref[...], preferred_element_type=jnp.float32) + bp_ref[...]
    )
    hw_ref[...] = jnp.dot(h, wt_ref[...], preferred_element_type=jnp.float32)


def _tc_mid(agg, xw, wb, b, wp, bp, wt):
    n = xw.shape[0]
    grid = (n // ROW_BLK,)
    return pl.pallas_call(
        _mid_body,
        grid=grid,
        in_specs=[
            pl.BlockSpec((ROW_BLK, AGG_DIM), lambda i: (i, 0)),
            pl.BlockSpec((ROW_BLK, AGG_DIM), lambda i: (i, 0)),
            pl.BlockSpec((AGG_DIM, AGG_DIM), lambda i: (0, 0)),
            pl.BlockSpec((AGG_DIM,), lambda i: (0,)),
            pl.BlockSpec((AGG_DIM, AGG_DIM), lambda i: (0, 0)),
            pl.BlockSpec((AGG_DIM,), lambda i: (0,)),
            pl.BlockSpec((AGG_DIM, AGG_DIM), lambda i: (0, 0)),
        ],
        out_specs=[
            pl.BlockSpec((ROW_BLK, AGG_DIM), lambda i: (i, 0)),
            pl.BlockSpec((ROW_BLK, AGG_DIM), lambda i: (i, 0)),
        ],
        out_shape=[
            jax.ShapeDtypeStruct((n, AGG_DIM), jnp.float32),
            jax.ShapeDtypeStruct((n, AGG_DIM), jnp.float32),
        ],
    )(agg, xw, wb, b, wp, bp, wt)


def _final_body(agg_ref, hw_ref, wb_ref, b_ref, wro_ref, bro_ref, out_ref):
    h = jax.nn.relu(
        hw_ref[...]
        + jnp.dot(agg_ref[...], wb_ref[...], preferred_element_type=jnp.float32)
        + b_ref[...]
    )
    sq = jnp.sum(h * h, axis=-1, keepdims=True)
    hn = h * lax.rsqrt(jnp.maximum(sq, 1e-12))
    out_ref[...] = jax.nn.relu(
        jnp.dot(hn, wro_ref[...], preferred_element_type=jnp.float32) + bro_ref[...]
    )


def _tc_final(agg, hw, wb, b, wro, bro):
    n = hw.shape[0]
    grid = (n // ROW_BLK,)
    return pl.pallas_call(
        _final_body,
        grid=grid,
        in_specs=[
            pl.BlockSpec((ROW_BLK, AGG_DIM), lambda i: (i, 0)),
            pl.BlockSpec((ROW_BLK, AGG_DIM), lambda i: (i, 0)),
            pl.BlockSpec((AGG_DIM, AGG_DIM), lambda i: (0, 0)),
            pl.BlockSpec((AGG_DIM,), lambda i: (0,)),
            pl.BlockSpec((AGG_DIM, RO_DIM), lambda i: (0, 0)),
            pl.BlockSpec((RO_DIM,), lambda i: (0,)),
        ],
        out_specs=pl.BlockSpec((ROW_BLK, RO_DIM), lambda i: (i, 0)),
        out_shape=jax.ShapeDtypeStruct((n, RO_DIM), jnp.float32),
    )(agg, hw, wb, b, wro, bro)


def _segmax_body(m_hbm, srcs_hbm, dsts_hbm, off_hbm, agg_hbm,
                 off_v, idx_v, dst_v, rows_v, acc_v, zrow_v, gsem):
    c = lax.axis_index("c")
    s = lax.axis_index("s")
    wid = s * 2 + c
    node_lo = wid * NPT

    pltpu.sync_copy(off_hbm, off_v)
    offs = off_v[pl.ds(wid, 16)]
    start = offs[0]
    end = offs[1]

    def _zf(i, _):
        zrow_v[i // SL, pl.ds((i % SL) * 16, 16)] = jnp.zeros((16,), jnp.float32)
        return 0
    lax.fori_loop(0, 64 * SL, _zf, 0)
    for j in range(NPT // 64):
        pltpu.sync_copy(zrow_v, agg_hbm.at[pl.ds(node_lo + j * 64, 64)])

    for sl in range(SL):
        acc_v[pl.ds(sl * 16, 16)] = jnp.zeros((16,), jnp.float32)

    abase = (start // 8) * 8
    nch = (end - abase + K - 1) // K

    def _chunk(ci, _):
        cbase = abase + ci * K
        pltpu.sync_copy(srcs_hbm.at[pl.ds(cbase, K)], idx_v)
        pltpu.sync_copy(dsts_hbm.at[pl.ds(cbase, K + 16)], dst_v)
        pltpu.async_copy(m_hbm.at[idx_v], rows_v, gsem).wait()
        e_lo = jnp.maximum(start, cbase) - cbase
        e_hi = jnp.minimum(end, cbase + K) - cbase

        def _edge(e, _):
            dpair = dst_v[pl.ds(e, 16)]
            d = dpair[0]
            nxt = dpair[1]
            for sl in range(SL):
                sli = pl.ds(sl * 16, 16)
                acc_v[sli] = jnp.maximum(acc_v[sli], rows_v[e, sli])

            @pl.when(nxt != d)
            def _flush():
                pltpu.sync_copy(acc_v, agg_hbm.at[d])
                for sl in range(SL):
                    acc_v[pl.ds(sl * 16, 16)] = jnp.zeros((16,), jnp.float32)
            return 0

        lax.fori_loop(e_lo, e_hi, _edge, 0)
        return 0

    lax.fori_loop(0, nch, _chunk, 0)


@functools.partial(
    pl.kernel,
    out_type=jax.ShapeDtypeStruct((NPAD, AGG_DIM), jnp.float32),
    mesh=plsc.VectorSubcoreMesh(core_axis_name="c", subcore_axis_name="s"),
    scratch_types=[
        pltpu.VMEM((48,), jnp.int32),
        pltpu.VMEM((K,), jnp.int32),
        pltpu.VMEM((K + 16,), jnp.int32),
        pltpu.VMEM((K, AGG_DIM), jnp.float32),
        pltpu.VMEM((AGG_DIM,), jnp.float32),
        pltpu.VMEM((64, AGG_DIM), jnp.float32),
        pltpu.SemaphoreType.DMA,
    ],
)
def _segmax_sc(m_hbm, srcs_hbm, dsts_hbm, off_hbm, agg_hbm,
               off_v, idx_v, dst_v, rows_v, acc_v, zrow_v, gsem):
    _segmax_body(m_hbm, srcs_hbm, dsts_hbm, off_hbm, agg_hbm,
                 off_v, idx_v, dst_v, rows_v, acc_v, zrow_v, gsem)


def kernel(x, edge_index, W_pool0, b_pool0, W0, b0, W_pool1, b_pool1, W1, b1, W_ro, b_ro):
    src = edge_index[0].astype(jnp.int32)
    dst = edge_index[1].astype(jnp.int32)
    dsts, srcs = lax.sort([dst, src], num_keys=1)
    bounds = (jnp.arange(NW + 1, dtype=jnp.int32) * NPT).astype(dst.dtype)
    off = jnp.searchsorted(dsts, bounds).astype(jnp.int32)
    off_p = jnp.concatenate([off, jnp.zeros((48 - (NW + 1),), jnp.int32)])
    srcs_p = jnp.concatenate([srcs, jnp.zeros((EPAD - N_EDGES,), jnp.int32)])
    dsts_p = jnp.concatenate(
        [dsts, jnp.full((EPAD - N_EDGES,), NPAD, jnp.int32)]
    )

    m0, xw0 = _tc_pool_and_top(x, W_pool0, b_pool0, W0[:D_FEAT])
    agg0 = _segmax_sc(m0, srcs_p, dsts_p, off_p)[:N_NODES]
    m1, hw1 = _tc_mid(agg0, xw0, W0[D_FEAT:], b0, W_pool1, b_pool1, W1[:AGG_DIM])
    agg1 = _segmax_sc(m1, srcs_p, dsts_p, off_p)[:N_NODES]
    return _tc_final(agg1, hw1, W1[AGG_DIM:], b1, W_ro, b_ro)
